# baseline (speedup 1.0000x reference)
import functools

import jax
import jax.numpy as jnp
from jax import lax
from jax.experimental import pallas as pl
from jax.experimental.pallas import tpu as pltpu
from jax.experimental.pallas import tpu_sc as plsc

_S = 8192
_D = 1024

_R_SC = 512
_R_TC = _S - _R_SC
_NC = 2
_NS = 16
_NW = _NC * _NS
_C = _R_SC // _NW

_mesh = plsc.VectorSubcoreMesh(core_axis_name="c", subcore_axis_name="s")


@functools.partial(
    pl.kernel,
    out_type=jax.ShapeDtypeStruct((_R_SC, _D), jnp.float32),
    mesh=_mesh,
    scratch_types=[
        pltpu.VMEM((1, _C), jnp.int32),
        pltpu.VMEM((_C, _D), jnp.float32),
        pltpu.VMEM((_C, _D), jnp.float32),
        pltpu.SemaphoreType.DMA,
    ],
)
def _posenc_sc(x_hbm, enc_hbm, out_hbm, idx_v, bufx, bufe, sem):
    wid = lax.axis_index("s") * _NC + lax.axis_index("c")
    obase = wid * _C
    wbase = _R_TC + obase
    idx_v[0, pl.ds(0, 16)] = wbase + lax.iota(jnp.int32, 16)

    cpx = pltpu.make_async_copy(x_hbm.at[pl.ds(wbase, _C)], bufx, sem)
    cpe = pltpu.make_async_copy(enc_hbm.at[idx_v.at[0]], bufe, sem)
    cpx.start()
    cpe.start()
    cpx.wait()
    cpe.wait()

    @pl.loop(0, _C)
    def _add_row(r):
        @pl.loop(0, _D, step=16, unroll=4)
        def _add_vec(c):
            s = pl.ds(c, 16)
            bufx[r, s] = bufx[r, s] + bufe[r, s]

    pltpu.sync_copy(bufx, out_hbm.at[pl.ds(obase, _C)])


_BLOCK_S = 512


def _add_block(x_ref, enc_ref, out_ref):
    out_ref[...] = x_ref[...] + enc_ref[...]


def _posenc_tc(x2, encoding):
    return pl.pallas_call(
        _add_block,
        grid=(_R_TC // _BLOCK_S,),
        in_specs=[
            pl.BlockSpec((_BLOCK_S, _D), lambda i: (i, 0)),
            pl.BlockSpec((_BLOCK_S, _D), lambda i: (i, 0)),
        ],
        out_specs=pl.BlockSpec((_BLOCK_S, _D), lambda i: (i, 0)),
        out_shape=jax.ShapeDtypeStruct((_S, _D), jnp.float32),
    )(x2, encoding)


def kernel(x, encoding):
    N, S, D = x.shape
    x2 = x.reshape(S, D)
    tc_out = _posenc_tc(x2, encoding)
    sc_out = _posenc_sc(x2, encoding)
    out = lax.dynamic_update_slice(tc_out, sc_out, (_R_TC, 0))
    return out.reshape(N, S, D)


# speedup vs baseline: 1.0172x; 1.0172x over previous
import functools

import jax
import jax.numpy as jnp
from jax import lax
from jax.experimental import pallas as pl
from jax.experimental.pallas import tpu as pltpu
from jax.experimental.pallas import tpu_sc as plsc

_S = 8192
_D = 1024

_R_SC = 512
_R_TC = _S - _R_SC
_NC = 2
_NS = 16
_NW = _NC * _NS
_C = _R_SC // _NW

_mesh = plsc.VectorSubcoreMesh(core_axis_name="c", subcore_axis_name="s")


@functools.partial(
    pl.kernel,
    out_type=jax.ShapeDtypeStruct((_R_SC, _D), jnp.float32),
    mesh=_mesh,
    scratch_types=[
        pltpu.VMEM((1, _C), jnp.int32),
        pltpu.VMEM((_C, _D), jnp.float32),
        pltpu.VMEM((_C, _D), jnp.float32),
        pltpu.SemaphoreType.DMA,
    ],
)
def _posenc_sc(x_hbm, enc_hbm, out_hbm, idx_v, bufx, bufe, sem):
    wid = lax.axis_index("s") * _NC + lax.axis_index("c")
    obase = wid * _C
    wbase = _R_TC + obase
    idx_v[0, pl.ds(0, 16)] = wbase + lax.iota(jnp.int32, 16)

    cpx = pltpu.make_async_copy(x_hbm.at[pl.ds(wbase, _C)], bufx, sem)
    cpe = pltpu.make_async_copy(enc_hbm.at[idx_v.at[0]], bufe, sem)
    cpx.start()
    cpe.start()
    cpx.wait()
    cpe.wait()

    @pl.loop(0, _C)
    def _add_row(r):
        @pl.loop(0, _D, step=16, unroll=4)
        def _add_vec(c):
            s = pl.ds(c, 16)
            bufx[r, s] = bufx[r, s] + bufe[r, s]

    pltpu.sync_copy(bufx, out_hbm.at[pl.ds(obase, _C)])


_BLOCK_S = 768


def _add_block(x_ref, enc_ref, out_ref):
    out_ref[...] = x_ref[...] + enc_ref[...]


def _posenc_tc(x2, encoding):
    return pl.pallas_call(
        _add_block,
        grid=(_R_TC // _BLOCK_S,),
        in_specs=[
            pl.BlockSpec((_BLOCK_S, _D), lambda i: (i, 0)),
            pl.BlockSpec((_BLOCK_S, _D), lambda i: (i, 0)),
        ],
        out_specs=pl.BlockSpec((_BLOCK_S, _D), lambda i: (i, 0)),
        out_shape=jax.ShapeDtypeStruct((_S, _D), jnp.float32),
    )(x2, encoding)


def kernel(x, encoding):
    N, S, D = x.shape
    x2 = x.reshape(S, D)
    tc_out = _posenc_tc(x2, encoding)
    sc_out = _posenc_sc(x2, encoding)
    out = lax.dynamic_update_slice(tc_out, sc_out, (_R_TC, 0))
    return out.reshape(N, S, D)


# speedup vs baseline: 1.0187x; 1.0015x over previous
import functools

import jax
import jax.numpy as jnp
from jax import lax
from jax.experimental import pallas as pl
from jax.experimental.pallas import tpu as pltpu
from jax.experimental.pallas import tpu_sc as plsc

_S = 8192
_D = 1024

_R_SC = 512
_R_TC = _S - _R_SC
_NC = 2
_NS = 16
_NW = _NC * _NS
_C = _R_SC // _NW

_mesh = plsc.VectorSubcoreMesh(core_axis_name="c", subcore_axis_name="s")


@functools.partial(
    pl.kernel,
    out_type=jax.ShapeDtypeStruct((_R_SC, _D), jnp.float32),
    mesh=_mesh,
    scratch_types=[
        pltpu.VMEM((1, _C), jnp.int32),
        pltpu.VMEM((_C, _D), jnp.float32),
        pltpu.VMEM((_C, _D), jnp.float32),
        pltpu.SemaphoreType.DMA,
    ],
)
def _posenc_sc(x_hbm, enc_hbm, out_hbm, idx_v, bufx, bufe, sem):
    wid = lax.axis_index("s") * _NC + lax.axis_index("c")
    obase = wid * _C
    wbase = _R_TC + obase
    idx_v[0, pl.ds(0, 16)] = wbase + lax.iota(jnp.int32, 16)

    cpx = pltpu.make_async_copy(x_hbm.at[pl.ds(wbase, _C)], bufx, sem)
    cpe = pltpu.make_async_copy(enc_hbm.at[idx_v.at[0]], bufe, sem)
    cpx.start()
    cpe.start()
    cpx.wait()
    cpe.wait()

    @pl.loop(0, _C)
    def _add_row(r):
        @pl.loop(0, _D, step=16, unroll=4)
        def _add_vec(c):
            s = pl.ds(c, 16)
            bufx[r, s] = bufx[r, s] + bufe[r, s]

    pltpu.sync_copy(bufx, out_hbm.at[pl.ds(obase, _C)])


_BLOCK_S = 960


def _add_block(x_ref, enc_ref, out_ref):
    out_ref[...] = x_ref[...] + enc_ref[...]


def _posenc_tc(x2, encoding):
    return pl.pallas_call(
        _add_block,
        grid=(_R_TC // _BLOCK_S,),
        in_specs=[
            pl.BlockSpec((_BLOCK_S, _D), lambda i: (i, 0)),
            pl.BlockSpec((_BLOCK_S, _D), lambda i: (i, 0)),
        ],
        out_specs=pl.BlockSpec((_BLOCK_S, _D), lambda i: (i, 0)),
        out_shape=jax.ShapeDtypeStruct((_S, _D), jnp.float32),
    )(x2, encoding)


def kernel(x, encoding):
    N, S, D = x.shape
    x2 = x.reshape(S, D)
    tc_out = _posenc_tc(x2, encoding)
    sc_out = _posenc_sc(x2, encoding)
    out = lax.dynamic_update_slice(tc_out, sc_out, (_R_TC, 0))
    return out.reshape(N, S, D)
